# baseline (speedup 1.0000x reference)
import functools

import jax
import jax.numpy as jnp
from jax.experimental import pallas as pl
from jax.experimental.pallas import tpu as pltpu

NUM_HEADS = 12
TOP_K = 2
NUM_EXPERTS = 8
D_MODEL = 768
D_FF = 1024
HEAD_DIM = D_MODEL // NUM_HEADS
SEQ = 2048

BQ = 2048
BK = 1024
BT = 256


def _attn_body(q_ref, k_ref, o_ref, m_ref, d_ref):
    kt = pl.program_id(2)

    @pl.when(kt == 0)
    def _():
        o_ref[0] = jnp.zeros((BQ, HEAD_DIM), jnp.float32)
        m_ref[...] = jnp.full((BQ, 1), -jnp.inf, jnp.float32)
        d_ref[...] = jnp.zeros((BQ, 1), jnp.float32)

    q = q_ref[0]
    k = k_ref[0]
    s = jax.lax.dot_general(q, k, (((1,), (1,)), ((), ())),
                            preferred_element_type=jnp.float32) * 0.125
    m_tile = jnp.max(s, axis=1, keepdims=True)
    m_old = m_ref[...]
    m_new = jnp.maximum(m_old, m_tile)
    delta = jnp.where(m_old == m_new, 0.0, m_old - m_new)
    p = jnp.exp(s - m_new)
    sum_tile = jnp.sum(p, axis=1, keepdims=True)
    scale = jnp.exp(delta)
    d_old = d_ref[...]
    coef = scale * d_old
    d_new = coef + sum_tile
    o_rescaled = coef * o_ref[0]
    o_unnorm = o_rescaled + jax.lax.dot_general(
        p, k, (((1,), (0,)), ((), ())), preferred_element_type=jnp.float32)
    o_ref[0] = o_unnorm * (1.0 / d_new)
    m_ref[...] = m_new
    d_ref[...] = d_new


def _attention(xh):
    grid = (NUM_HEADS, SEQ // BQ, SEQ // BK)
    return pl.pallas_call(
        _attn_body,
        grid=grid,
        in_specs=[
            pl.BlockSpec((1, BQ, HEAD_DIM), lambda h, q, kt: (h, q, 0)),
            pl.BlockSpec((1, BK, HEAD_DIM), lambda h, q, kt: (h, kt, 0)),
        ],
        out_specs=pl.BlockSpec((1, BQ, HEAD_DIM), lambda h, q, kt: (h, q, 0)),
        out_shape=jax.ShapeDtypeStruct((NUM_HEADS, SEQ, HEAD_DIM),
                                       jnp.float32),
        scratch_shapes=[pltpu.VMEM((BQ, 1), jnp.float32),
                        pltpu.VMEM((BQ, 1), jnp.float32)],
        compiler_params=pltpu.CompilerParams(
            dimension_semantics=("parallel", "parallel", "arbitrary")),
    )(xh, xh)


def _gate_body(xf_ref, wg_ref, fw_ref):
    xf = xf_ref[...]
    wg = wg_ref[...]
    logits = jax.lax.dot_general(xf, wg, (((1,), (1,)), ((), ())),
                                 preferred_element_type=jnp.float32)
    idx = jax.lax.broadcasted_iota(jnp.int32, logits.shape, 1)
    m1 = jnp.max(logits, axis=1, keepdims=True)
    i1 = jnp.min(jnp.where(logits == m1, idx, NUM_EXPERTS), axis=1,
                 keepdims=True)
    masked = jnp.where(idx == i1, -jnp.inf, logits)
    m2 = jnp.max(masked, axis=1, keepdims=True)
    i2 = jnp.min(jnp.where(masked == m2, idx, NUM_EXPERTS), axis=1,
                 keepdims=True)
    w1 = 1.0 / (1.0 + jnp.exp(m2 - m1))
    w2 = 1.0 - w1
    fw_ref[...] = (jnp.where(idx == i1, w1, 0.0)
                   + jnp.where(idx == i2, w2, 0.0))


def _gating(xf, Wg):
    return pl.pallas_call(
        _gate_body,
        out_shape=jax.ShapeDtypeStruct((SEQ, NUM_EXPERTS), jnp.float32),
    )(xf, Wg)


def _moe_body(xf_ref, fw_ref, w1_ref, b1_ref, w2_ref, b2_ref, o_ref):
    xf = xf_ref[...].astype(jnp.bfloat16)
    fw = fw_ref[...]
    lane = jax.lax.broadcasted_iota(jnp.int32, fw.shape, 1)
    acc = jnp.zeros((BT, D_MODEL), jnp.float32)
    for e in range(NUM_EXPERTS):
        w1 = w1_ref[e].astype(jnp.bfloat16)
        h = jax.lax.dot_general(xf, w1, (((1,), (1,)), ((), ())),
                                preferred_element_type=jnp.float32)
        h = h + b1_ref[e][None, :]
        h = (h * 0.5 * (1.0 + jax.lax.erf(h * 0.7071067811865476))
             ).astype(jnp.bfloat16)
        w2 = w2_ref[e].astype(jnp.bfloat16)
        eo = jax.lax.dot_general(h, w2, (((1,), (1,)), ((), ())),
                                 preferred_element_type=jnp.float32)
        eo = eo + b2_ref[e][None, :]
        wcol = jnp.sum(jnp.where(lane == e, fw, 0.0), axis=1, keepdims=True)
        acc = acc + eo * wcol
    o_ref[...] = acc


def _moe_dense(xf, fw, W1, b1, W2, b2):
    grid = (SEQ // BT,)
    return pl.pallas_call(
        _moe_body,
        grid=grid,
        in_specs=[
            pl.BlockSpec((BT, D_MODEL), lambda t: (t, 0)),
            pl.BlockSpec((BT, NUM_EXPERTS), lambda t: (t, 0)),
            pl.BlockSpec((NUM_EXPERTS, D_FF, D_MODEL), lambda t: (0, 0, 0)),
            pl.BlockSpec((NUM_EXPERTS, D_FF), lambda t: (0, 0)),
            pl.BlockSpec((NUM_EXPERTS, D_MODEL, D_FF), lambda t: (0, 0, 0)),
            pl.BlockSpec((NUM_EXPERTS, D_MODEL), lambda t: (0, 0)),
        ],
        out_specs=pl.BlockSpec((BT, D_MODEL), lambda t: (t, 0)),
        out_shape=jax.ShapeDtypeStruct((SEQ, D_MODEL), jnp.float32),
        compiler_params=pltpu.CompilerParams(
            dimension_semantics=("arbitrary",)),
    )(xf, fw, W1, b1, W2, b2)


BLK = 128
CAP = 5120
NT = CAP // BLK
CB = 256


def _route_body(xf_ref, wg_ref, pos1_ref, pos2_ref, w1o_ref, w2o_ref,
                cnt_ref):
    xf = xf_ref[...]
    wg = wg_ref[...]
    logits = jax.lax.dot_general(xf, wg, (((1,), (1,)), ((), ())),
                                 preferred_element_type=jnp.float32)
    idx = jax.lax.broadcasted_iota(jnp.int32, logits.shape, 1)
    m1 = jnp.max(logits, axis=1, keepdims=True)
    i1 = jnp.min(jnp.where(logits == m1, idx, NUM_EXPERTS), axis=1,
                 keepdims=True)
    masked = jnp.where(idx == i1, -jnp.inf, logits)
    m2 = jnp.max(masked, axis=1, keepdims=True)
    i2 = jnp.min(jnp.where(masked == m2, idx, NUM_EXPERTS), axis=1,
                 keepdims=True)
    w1o_ref[...] = 1.0 / (1.0 + jnp.exp(m2 - m1))
    w2o_ref[...] = 1.0 - w1o_ref[...]
    ind = ((idx == i1) | (idx == i2)).astype(jnp.float32)

    ri = jax.lax.broadcasted_iota(jnp.int32, (CB, CB), 0)
    ci = jax.lax.broadcasted_iota(jnp.int32, (CB, CB), 1)
    tril = (ci < ri).astype(jnp.float32)
    carry = jnp.zeros((1, NUM_EXPERTS), jnp.float32)
    ranks_parts = []
    for b in range(SEQ // CB):
        blk = ind[b * CB:(b + 1) * CB, :]
        r = jax.lax.dot_general(tril, blk, (((1,), (0,)), ((), ())),
                                preferred_element_type=jnp.float32)
        ranks_parts.append(r + carry)
        carry = carry + jnp.sum(blk, axis=0, keepdims=True)
    ranks = jnp.concatenate(ranks_parts, axis=0)
    cnt_ref[...] = carry

    padded = jnp.ceil(carry * (1.0 / BLK)) * BLK
    pb = jnp.broadcast_to(padded, (NUM_EXPERTS, NUM_EXPERTS))
    ei = jax.lax.broadcasted_iota(jnp.int32, (NUM_EXPERTS, NUM_EXPERTS), 0)
    ej = jax.lax.broadcasted_iota(jnp.int32, (NUM_EXPERTS, NUM_EXPERTS), 1)
    mlt = (ei < ej).astype(jnp.float32)
    base_all = jax.lax.dot_general(pb, mlt, (((1,), (0,)), ((), ())),
                                   preferred_element_type=jnp.float32)
    base = base_all[0:1, :]
    pos = base + ranks
    pos1_ref[...] = jnp.sum(jnp.where(idx == i1, pos, 0.0), axis=1,
                            keepdims=True)
    pos2_ref[...] = jnp.sum(jnp.where(idx == i2, pos, 0.0), axis=1,
                            keepdims=True)


def _routing(xf, Wg):
    outs = pl.pallas_call(
        _route_body,
        out_shape=[
            jax.ShapeDtypeStruct((SEQ, 1), jnp.float32),
            jax.ShapeDtypeStruct((SEQ, 1), jnp.float32),
            jax.ShapeDtypeStruct((SEQ, 1), jnp.float32),
            jax.ShapeDtypeStruct((SEQ, 1), jnp.float32),
            jax.ShapeDtypeStruct((1, NUM_EXPERTS), jnp.float32),
        ],
    )(xf, Wg)
    return outs


from jax import lax as _lax
from jax.experimental.pallas import tpu_sc as plsc

_SC_INFO = plsc.get_sparse_core_info()
_NW = _SC_INFO.num_cores * _SC_INFO.num_subcores
_TPW = SEQ // _NW


def _sc_scatter(xf, pos1, pos2):
    mesh = plsc.VectorSubcoreMesh(core_axis_name="c", subcore_axis_name="s")

    @functools.partial(
        pl.kernel, mesh=mesh,
        out_type=jax.ShapeDtypeStruct((CAP, D_MODEL), jnp.float32),
        scratch_types=[
            pltpu.VMEM((_TPW,), jnp.int32),
            pltpu.VMEM((_TPW,), jnp.int32),
            pltpu.VMEM((_TPW, D_MODEL), jnp.float32),
            pltpu.SemaphoreType.DMA,
        ],
    )
    def k(xf_hbm, p1_hbm, p2_hbm, y_hbm, i1_v, i2_v, rows_v, sem):
        wid = _lax.axis_index("s") * _SC_INFO.num_cores + _lax.axis_index("c")
        base = wid * _TPW
        pltpu.sync_copy(p1_hbm.at[pl.ds(base, _TPW)], i1_v)
        pltpu.sync_copy(p2_hbm.at[pl.ds(base, _TPW)], i2_v)
        pltpu.sync_copy(xf_hbm.at[pl.ds(base, _TPW)], rows_v)
        pltpu.async_copy(rows_v, y_hbm.at[i1_v], sem).wait()
        pltpu.async_copy(rows_v, y_hbm.at[i2_v], sem).wait()

    return k(xf, pos1, pos2)


def _sc_gather_combine(contrib, pos1, pos2, w1c, w2c):
    mesh = plsc.VectorSubcoreMesh(core_axis_name="c", subcore_axis_name="s")
    nvec = D_MODEL // 16

    @functools.partial(
        pl.kernel, mesh=mesh,
        out_type=jax.ShapeDtypeStruct((SEQ, D_MODEL), jnp.float32),
        scratch_types=[
            pltpu.VMEM((_TPW,), jnp.int32),
            pltpu.VMEM((_TPW, D_MODEL), jnp.float32),
            pltpu.VMEM((_TPW, D_MODEL), jnp.float32),
            pltpu.VMEM((_TPW,), jnp.float32),
            pltpu.VMEM((_TPW,), jnp.float32),
            pltpu.SemaphoreType.DMA,
        ],
    )
    def k(c_hbm, p1_hbm, p2_hbm, wa_hbm, wb_hbm, o_hbm,
          i_v, ba_v, bb_v, wa_v, wb_v, sem):
        wid = _lax.axis_index("s") * _SC_INFO.num_cores + _lax.axis_index("c")
        base = wid * _TPW
        pltpu.sync_copy(p1_hbm.at[pl.ds(base, _TPW)], i_v)
        pltpu.async_copy(c_hbm.at[i_v], ba_v, sem).wait()
        pltpu.sync_copy(p2_hbm.at[pl.ds(base, _TPW)], i_v)
        pltpu.async_copy(c_hbm.at[i_v], bb_v, sem).wait()
        pltpu.sync_copy(wa_hbm.at[pl.ds(base, _TPW)], wa_v)
        pltpu.sync_copy(wb_hbm.at[pl.ds(base, _TPW)], wb_v)

        def grp(g, carry):
            wa16 = wa_v[pl.ds(g * 16, 16)]
            wb16 = wb_v[pl.ds(g * 16, 16)]
            for r in range(16):
                wa = wa16[r]
                wb = wb16[r]
                i = g * 16 + r
                for j in range(nvec):
                    sl = pl.ds(j * 16, 16)
                    ba_v[i, sl] = wa * ba_v[i, sl] + wb * bb_v[i, sl]
            return carry
        _lax.fori_loop(0, _TPW // 16, grp, 0)
        pltpu.sync_copy(ba_v, o_hbm.at[pl.ds(base, _TPW)])

    return k(contrib, pos1, pos2, w1c, w2c)


def _gemm_body(te_ref, y_ref, w1_ref, b1_ref, w2_ref, b2_ref, o_ref):
    y = y_ref[...].astype(jnp.bfloat16)
    w1 = w1_ref[0].astype(jnp.bfloat16)
    h = jax.lax.dot_general(y, w1, (((1,), (1,)), ((), ())),
                            preferred_element_type=jnp.float32)
    h = h + b1_ref[0]
    h = (h * 0.5 * (1.0 + jax.lax.erf(h * 0.7071067811865476))
         ).astype(jnp.bfloat16)
    w2 = w2_ref[0].astype(jnp.bfloat16)
    eo = jax.lax.dot_general(h, w2, (((1,), (1,)), ((), ())),
                             preferred_element_type=jnp.float32)
    o_ref[...] = eo + b2_ref[0]


def _grouped_gemm(y, te, W1, b1, W2, b2):
    grid_spec = pltpu.PrefetchScalarGridSpec(
        num_scalar_prefetch=1,
        grid=(NT,),
        in_specs=[
            pl.BlockSpec((BLK, D_MODEL), lambda t, te: (t, 0)),
            pl.BlockSpec((1, D_FF, D_MODEL), lambda t, te: (te[t], 0, 0)),
            pl.BlockSpec((1, 1, D_FF), lambda t, te: (te[t], 0, 0)),
            pl.BlockSpec((1, D_MODEL, D_FF), lambda t, te: (te[t], 0, 0)),
            pl.BlockSpec((1, 1, D_MODEL), lambda t, te: (te[t], 0, 0)),
        ],
        out_specs=pl.BlockSpec((BLK, D_MODEL), lambda t, te: (t, 0)),
    )
    return pl.pallas_call(
        _gemm_body,
        grid_spec=grid_spec,
        out_shape=jax.ShapeDtypeStruct((CAP, D_MODEL), jnp.float32),
        compiler_params=pltpu.CompilerParams(
            dimension_semantics=("arbitrary",)),
    )(te, y, W1, b1, W2, b2)


def _combine_body(g1_ref, g2_ref, w1_ref, w2_ref, o_ref):
    o_ref[...] = w1_ref[...] * g1_ref[...] + w2_ref[...] * g2_ref[...]


def _combine(g1, g2, w1c, w2c):
    grid = (SEQ // BT,)
    return pl.pallas_call(
        _combine_body,
        grid=grid,
        in_specs=[
            pl.BlockSpec((BT, D_MODEL), lambda t: (t, 0)),
            pl.BlockSpec((BT, D_MODEL), lambda t: (t, 0)),
            pl.BlockSpec((BT, 1), lambda t: (t, 0)),
            pl.BlockSpec((BT, 1), lambda t: (t, 0)),
        ],
        out_specs=pl.BlockSpec((BT, D_MODEL), lambda t: (t, 0)),
        out_shape=jax.ShapeDtypeStruct((SEQ, D_MODEL), jnp.float32),
    )(g1, g2, w1c, w2c)


def _moe_routed(xf, Wg, W1, b1, W2, b2):
    pos1f, pos2f, w1c, w2c, counts = _routing(xf, Wg)
    pos1 = pos1f.astype(jnp.int32).reshape(SEQ)
    pos2 = pos2f.astype(jnp.int32).reshape(SEQ)
    padded = jnp.ceil(counts[0] * (1.0 / BLK)).astype(jnp.int32) * BLK
    base = jnp.concatenate([jnp.zeros((1,), jnp.int32),
                            jnp.cumsum(padded)[:-1]])
    te = (jnp.sum(base[None, :] <= (jnp.arange(NT) * BLK)[:, None],
                  axis=1) - 1).astype(jnp.int32)
    y = _sc_scatter(xf, pos1, pos2)
    contrib = _grouped_gemm(y, te, W1, b1.reshape(NUM_EXPERTS, 1, D_FF),
                            W2, b2.reshape(NUM_EXPERTS, 1, D_MODEL))
    return _sc_gather_combine(contrib, pos1, pos2,
                              w1c.reshape(SEQ), w2c.reshape(SEQ))


@jax.jit
def kernel(x, Wg, W1, b1, W2, b2):
    xh = x.reshape(SEQ, NUM_HEADS, HEAD_DIM).transpose(1, 0, 2)
    attn = _attention(xh).transpose(1, 0, 2).reshape(SEQ, D_MODEL)
    out = _moe_routed(attn, Wg, W1, b1, W2, b2)
    return out.reshape(1, SEQ, D_MODEL)


# speedup vs baseline: 1.0203x; 1.0203x over previous
import functools

import jax
import jax.numpy as jnp
from jax.experimental import pallas as pl
from jax.experimental.pallas import tpu as pltpu

NUM_HEADS = 12
TOP_K = 2
NUM_EXPERTS = 8
D_MODEL = 768
D_FF = 1024
HEAD_DIM = D_MODEL // NUM_HEADS
SEQ = 2048

BQ = 2048
BK = 1024
BT = 256


def _attn_body(q_ref, k_ref, o_ref, m_ref, d_ref):
    kt = pl.program_id(2)

    @pl.when(kt == 0)
    def _():
        o_ref[0] = jnp.zeros((BQ, HEAD_DIM), jnp.float32)
        m_ref[...] = jnp.full((BQ, 1), -jnp.inf, jnp.float32)
        d_ref[...] = jnp.zeros((BQ, 1), jnp.float32)

    q = q_ref[0]
    k = k_ref[0]
    s = jax.lax.dot_general(q, k, (((1,), (1,)), ((), ())),
                            preferred_element_type=jnp.float32) * 0.125
    m_tile = jnp.max(s, axis=1, keepdims=True)
    m_old = m_ref[...]
    m_new = jnp.maximum(m_old, m_tile)
    delta = jnp.where(m_old == m_new, 0.0, m_old - m_new)
    p = jnp.exp(s - m_new)
    sum_tile = jnp.sum(p, axis=1, keepdims=True)
    scale = jnp.exp(delta)
    d_old = d_ref[...]
    coef = scale * d_old
    d_new = coef + sum_tile
    o_rescaled = coef * o_ref[0]
    o_unnorm = o_rescaled + jax.lax.dot_general(
        p, k, (((1,), (0,)), ((), ())), preferred_element_type=jnp.float32)
    o_ref[0] = o_unnorm * (1.0 / d_new)
    m_ref[...] = m_new
    d_ref[...] = d_new


def _attention(xh):
    grid = (NUM_HEADS, SEQ // BQ, SEQ // BK)
    return pl.pallas_call(
        _attn_body,
        grid=grid,
        in_specs=[
            pl.BlockSpec((1, BQ, HEAD_DIM), lambda h, q, kt: (h, q, 0)),
            pl.BlockSpec((1, BK, HEAD_DIM), lambda h, q, kt: (h, kt, 0)),
        ],
        out_specs=pl.BlockSpec((1, BQ, HEAD_DIM), lambda h, q, kt: (h, q, 0)),
        out_shape=jax.ShapeDtypeStruct((NUM_HEADS, SEQ, HEAD_DIM),
                                       jnp.float32),
        scratch_shapes=[pltpu.VMEM((BQ, 1), jnp.float32),
                        pltpu.VMEM((BQ, 1), jnp.float32)],
        compiler_params=pltpu.CompilerParams(
            dimension_semantics=("parallel", "parallel", "arbitrary")),
    )(xh, xh)


BLK = 128
CAP = 5120
NT = CAP // BLK
CB = 256


def _route_body(xf_ref, wg_ref, pos1_ref, pos2_ref, w1o_ref, w2o_ref,
                cnt_ref):
    xf = xf_ref[...]
    wg = wg_ref[...]
    logits = jax.lax.dot_general(xf, wg, (((1,), (1,)), ((), ())),
                                 preferred_element_type=jnp.float32)
    idx = jax.lax.broadcasted_iota(jnp.int32, logits.shape, 1)
    m1 = jnp.max(logits, axis=1, keepdims=True)
    i1 = jnp.min(jnp.where(logits == m1, idx, NUM_EXPERTS), axis=1,
                 keepdims=True)
    masked = jnp.where(idx == i1, -jnp.inf, logits)
    m2 = jnp.max(masked, axis=1, keepdims=True)
    i2 = jnp.min(jnp.where(masked == m2, idx, NUM_EXPERTS), axis=1,
                 keepdims=True)
    w1o_ref[...] = 1.0 / (1.0 + jnp.exp(m2 - m1))
    w2o_ref[...] = 1.0 - w1o_ref[...]
    ind = ((idx == i1) | (idx == i2)).astype(jnp.float32)

    ri = jax.lax.broadcasted_iota(jnp.int32, (CB, CB), 0)
    ci = jax.lax.broadcasted_iota(jnp.int32, (CB, CB), 1)
    tril = (ci < ri).astype(jnp.float32)
    carry = jnp.zeros((1, NUM_EXPERTS), jnp.float32)
    ranks_parts = []
    for b in range(SEQ // CB):
        blk = ind[b * CB:(b + 1) * CB, :]
        r = jax.lax.dot_general(tril, blk, (((1,), (0,)), ((), ())),
                                preferred_element_type=jnp.float32)
        ranks_parts.append(r + carry)
        carry = carry + jnp.sum(blk, axis=0, keepdims=True)
    ranks = jnp.concatenate(ranks_parts, axis=0)
    cnt_ref[...] = carry

    padded = jnp.ceil(carry * (1.0 / BLK)) * BLK
    pb = jnp.broadcast_to(padded, (NUM_EXPERTS, NUM_EXPERTS))
    ei = jax.lax.broadcasted_iota(jnp.int32, (NUM_EXPERTS, NUM_EXPERTS), 0)
    ej = jax.lax.broadcasted_iota(jnp.int32, (NUM_EXPERTS, NUM_EXPERTS), 1)
    mlt = (ei < ej).astype(jnp.float32)
    base_all = jax.lax.dot_general(pb, mlt, (((1,), (0,)), ((), ())),
                                   preferred_element_type=jnp.float32)
    base = base_all[0:1, :]
    pos = base + ranks
    pos1_ref[...] = jnp.sum(jnp.where(idx == i1, pos, 0.0), axis=1,
                            keepdims=True)
    pos2_ref[...] = jnp.sum(jnp.where(idx == i2, pos, 0.0), axis=1,
                            keepdims=True)


def _routing(xf, Wg):
    outs = pl.pallas_call(
        _route_body,
        out_shape=[
            jax.ShapeDtypeStruct((SEQ, 1), jnp.float32),
            jax.ShapeDtypeStruct((SEQ, 1), jnp.float32),
            jax.ShapeDtypeStruct((SEQ, 1), jnp.float32),
            jax.ShapeDtypeStruct((SEQ, 1), jnp.float32),
            jax.ShapeDtypeStruct((1, NUM_EXPERTS), jnp.float32),
        ],
    )(xf, Wg)
    return outs


from jax import lax as _lax
from jax.experimental.pallas import tpu_sc as plsc

_SC_INFO = plsc.get_sparse_core_info()
_NW = _SC_INFO.num_cores * _SC_INFO.num_subcores
_TPW = SEQ // _NW


def _sc_scatter(xf, pos1, pos2):
    mesh = plsc.VectorSubcoreMesh(core_axis_name="c", subcore_axis_name="s")

    @functools.partial(
        pl.kernel, mesh=mesh,
        out_type=jax.ShapeDtypeStruct((CAP, D_MODEL), jnp.float32),
        scratch_types=[
            pltpu.VMEM((_TPW,), jnp.int32),
            pltpu.VMEM((_TPW,), jnp.int32),
            pltpu.VMEM((_TPW, D_MODEL), jnp.float32),
            pltpu.SemaphoreType.DMA,
        ],
    )
    def k(xf_hbm, p1_hbm, p2_hbm, y_hbm, i1_v, i2_v, rows_v, sem):
        wid = _lax.axis_index("s") * _SC_INFO.num_cores + _lax.axis_index("c")
        base = wid * _TPW
        pltpu.sync_copy(p1_hbm.at[pl.ds(base, _TPW)], i1_v)
        pltpu.sync_copy(p2_hbm.at[pl.ds(base, _TPW)], i2_v)
        pltpu.sync_copy(xf_hbm.at[pl.ds(base, _TPW)], rows_v)
        pltpu.async_copy(rows_v, y_hbm.at[i1_v], sem).wait()
        pltpu.async_copy(rows_v, y_hbm.at[i2_v], sem).wait()

    return k(xf, pos1, pos2)


def _sc_gather(contrib, pos1, pos2):
    mesh = plsc.VectorSubcoreMesh(core_axis_name="c", subcore_axis_name="s")

    @functools.partial(
        pl.kernel, mesh=mesh,
        out_type=[jax.ShapeDtypeStruct((SEQ, D_MODEL), jnp.float32),
                  jax.ShapeDtypeStruct((SEQ, D_MODEL), jnp.float32)],
        scratch_types=[
            pltpu.VMEM((_TPW,), jnp.int32),
            pltpu.VMEM((_TPW, D_MODEL), jnp.float32),
            pltpu.SemaphoreType.DMA,
        ],
    )
    def k(c_hbm, p1_hbm, p2_hbm, g1_hbm, g2_hbm, i_v, rows_v, sem):
        wid = _lax.axis_index("s") * _SC_INFO.num_cores + _lax.axis_index("c")
        base = wid * _TPW
        pltpu.sync_copy(p1_hbm.at[pl.ds(base, _TPW)], i_v)
        pltpu.async_copy(c_hbm.at[i_v], rows_v, sem).wait()
        pltpu.sync_copy(rows_v, g1_hbm.at[pl.ds(base, _TPW)])
        pltpu.sync_copy(p2_hbm.at[pl.ds(base, _TPW)], i_v)
        pltpu.async_copy(c_hbm.at[i_v], rows_v, sem).wait()
        pltpu.sync_copy(rows_v, g2_hbm.at[pl.ds(base, _TPW)])

    return k(contrib, pos1, pos2)


def _gemm_body(te_ref, y_ref, w1_ref, b1_ref, w2_ref, b2_ref, o_ref):
    y = y_ref[...].astype(jnp.bfloat16)
    w1 = w1_ref[0].astype(jnp.bfloat16)
    h = jax.lax.dot_general(y, w1, (((1,), (1,)), ((), ())),
                            preferred_element_type=jnp.float32)
    h = h + b1_ref[0]
    h = (h * 0.5 * (1.0 + jax.lax.erf(h * 0.7071067811865476))
         ).astype(jnp.bfloat16)
    w2 = w2_ref[0].astype(jnp.bfloat16)
    eo = jax.lax.dot_general(h, w2, (((1,), (1,)), ((), ())),
                             preferred_element_type=jnp.float32)
    o_ref[...] = eo + b2_ref[0]


def _grouped_gemm(y, te, W1, b1, W2, b2):
    grid_spec = pltpu.PrefetchScalarGridSpec(
        num_scalar_prefetch=1,
        grid=(NT,),
        in_specs=[
            pl.BlockSpec((BLK, D_MODEL), lambda t, te: (t, 0)),
            pl.BlockSpec((1, D_FF, D_MODEL), lambda t, te: (te[t], 0, 0)),
            pl.BlockSpec((1, 1, D_FF), lambda t, te: (te[t], 0, 0)),
            pl.BlockSpec((1, D_MODEL, D_FF), lambda t, te: (te[t], 0, 0)),
            pl.BlockSpec((1, 1, D_MODEL), lambda t, te: (te[t], 0, 0)),
        ],
        out_specs=pl.BlockSpec((BLK, D_MODEL), lambda t, te: (t, 0)),
    )
    return pl.pallas_call(
        _gemm_body,
        grid_spec=grid_spec,
        out_shape=jax.ShapeDtypeStruct((CAP, D_MODEL), jnp.float32),
        compiler_params=pltpu.CompilerParams(
            dimension_semantics=("arbitrary",)),
    )(te, y, W1, b1, W2, b2)


def _combine_body(g1_ref, g2_ref, w1_ref, w2_ref, o_ref):
    o_ref[...] = w1_ref[...] * g1_ref[...] + w2_ref[...] * g2_ref[...]


def _combine(g1, g2, w1c, w2c):
    grid = (SEQ // BT,)
    return pl.pallas_call(
        _combine_body,
        grid=grid,
        in_specs=[
            pl.BlockSpec((BT, D_MODEL), lambda t: (t, 0)),
            pl.BlockSpec((BT, D_MODEL), lambda t: (t, 0)),
            pl.BlockSpec((BT, 1), lambda t: (t, 0)),
            pl.BlockSpec((BT, 1), lambda t: (t, 0)),
        ],
        out_specs=pl.BlockSpec((BT, D_MODEL), lambda t: (t, 0)),
        out_shape=jax.ShapeDtypeStruct((SEQ, D_MODEL), jnp.float32),
    )(g1, g2, w1c, w2c)


def _moe_routed(xf, Wg, W1, b1, W2, b2):
    pos1f, pos2f, w1c, w2c, counts = _routing(xf, Wg)
    pos1 = pos1f.astype(jnp.int32).reshape(SEQ)
    pos2 = pos2f.astype(jnp.int32).reshape(SEQ)
    padded = jnp.ceil(counts[0] * (1.0 / BLK)).astype(jnp.int32) * BLK
    base = jnp.concatenate([jnp.zeros((1,), jnp.int32),
                            jnp.cumsum(padded)[:-1]])
    te = (jnp.sum(base[None, :] <= (jnp.arange(NT) * BLK)[:, None],
                  axis=1) - 1).astype(jnp.int32)
    y = _sc_scatter(xf, pos1, pos2)
    contrib = _grouped_gemm(y, te, W1, b1.reshape(NUM_EXPERTS, 1, D_FF),
                            W2, b2.reshape(NUM_EXPERTS, 1, D_MODEL))
    g1, g2 = _sc_gather(contrib, pos1, pos2)
    return _combine(g1, g2, w1c, w2c)


@jax.jit
def kernel(x, Wg, W1, b1, W2, b2):
    xh = x.reshape(SEQ, NUM_HEADS, HEAD_DIM).transpose(1, 0, 2)
    attn = _attention(xh).transpose(1, 0, 2).reshape(SEQ, D_MODEL)
    out = _moe_routed(attn, Wg, W1, b1, W2, b2)
    return out.reshape(1, SEQ, D_MODEL)
